# baseline (speedup 1.0000x reference)
import functools

import jax
import jax.numpy as jnp
from jax import lax
from jax.experimental import pallas as pl
from jax.experimental.pallas import tpu as pltpu
from jax.experimental.pallas import tpu_sc as plsc

_N = 10000
_M = 320000
_C = 2000
_FEA = 128
_NG = 3
_HEADS = 3
_HID = 256

_NC = 2
_NS = 16
_NW = _NC * _NS
_WPT = 2
_NV = _NW * _WPT
_NPW = 160
_NOUT = _NV * _NPW
_CH = 16
_GRP = 256
_MP = _M + 2 * _GRP
_TNW = 2 * _HID + 128
_DR = _NPW // 16


def _sc_edge_body(ts_hbm, tn_hbm, s2_hbm, n2_hbm, w2g_hbm, est_hbm,
                  acc_hbm, den_hbm,
                  sblk, nblk, tsb0, tsb1, tnb0, tnb1, tile, denb, w2gb,
                  estb, gbuf, sems):
    cid = lax.axis_index("c")
    sid = lax.axis_index("s")
    w = sid * _NC + cid

    pltpu.sync_copy(w2g_hbm, w2gb)
    zv = jnp.zeros((16,), jnp.float32)
    lanes = lax.iota(jnp.int32, 16)

    def _pass(sp, _0):
        wid = w * _WPT + sp
        nbase = wid * _NPW
        pltpu.sync_copy(est_hbm.at[wid], estb)
        ev = estb[0, pl.ds(0, 16)]
        e0 = ev[0]
        e1 = ev[1]

        def _zrow(r, _):
            for k2 in range(_HID // 16):
                tile[r, pl.ds(k2 * 16, 16)] = zv
            return 0
        lax.fori_loop(0, _NPW, _zrow, 0)

        def _zden(r, _):
            denb[r, pl.ds(0, 16)] = zv
            return 0
        lax.fori_loop(0, _DR, _zden, 0)

        e0a = (e0 // _GRP) * _GRP
        ng = (e1 - e0a + _GRP - 1) // _GRP

        def _chunk(k, tsr, tnr, base):
            cbase = base + k * _CH
            sv = sblk[k, pl.ds(0, 16)]
            for e in range(_CH):
                lv = tnr[e, pl.ds(2 * _HID, 16)]
                lv = jnp.where(lv > 0, lv, 0.01 * lv)

                def _dot4(q, vp):
                    for t in range(4):
                        k2 = q * 4 + t
                        a = (tsr[e, pl.ds(k2 * 16, 16)]
                             + tnr[e, pl.ds(k2 * 16, 16)])
                        a = jnp.where(a > 0, a, 0.01 * a)
                        vp = vp + a * w2gb[pl.ds(k2 * 16, 16)]
                    return vp
                vp = lax.fori_loop(0, _HID // 64, _dot4,
                                   lv * w2gb[pl.ds(_HID, 16)])
                gbuf[pl.ds(e * _CH, _CH)] = vp

            gvec = zv
            for j in range(_CH):
                gvec = gvec + plsc.load_gather(gbuf, [lanes * _CH + j])

            mask = (lanes + cbase >= e0) & (lanes + cbase < e1)
            uv = jnp.where(mask, jnp.exp(gvec), 0.0)

            for e in range(_CH):
                u = uv[e]
                sl = jnp.clip(sv[e] - nbase, 0, _NPW - 1)

                def _acc4(q, _2):
                    for t in range(4):
                        k2 = q * 4 + t
                        bvec = (tsr[e, pl.ds(_HID + k2 * 16, 16)]
                                + tnr[e, pl.ds(_HID + k2 * 16, 16)])
                        bvec = jnp.where(bvec > 0, bvec, 0.01 * bvec)
                        tile[sl, pl.ds(k2 * 16, 16)] = (
                            tile[sl, pl.ds(k2 * 16, 16)] + u * bvec)
                    return 0
                lax.fori_loop(0, _HID // 64, _acc4, 0)
                dr = sl // 16
                dhot = jnp.where(lanes == sl % 16, u, 0.0)
                denb[dr, pl.ds(0, 16)] = denb[dr, pl.ds(0, 16)] + dhot

        def _group(g, _):
            base = e0a + g * _GRP
            gi = base // _GRP
            pltpu.sync_copy(s2_hbm.at[gi], sblk)
            pltpu.sync_copy(n2_hbm.at[gi], nblk)

            def _pair(cp, _2):
                k0 = 2 * cp
                k1 = k0 + 1
                h0 = (pltpu.async_copy(ts_hbm.at[sblk.at[k0]], tsb0,
                                       sems.at[0]),
                      pltpu.async_copy(tn_hbm.at[nblk.at[k0]], tnb0,
                                       sems.at[1]))
                h1 = (pltpu.async_copy(ts_hbm.at[sblk.at[k1]], tsb1,
                                       sems.at[2]),
                      pltpu.async_copy(tn_hbm.at[nblk.at[k1]], tnb1,
                                       sems.at[3]))
                h0[0].wait()
                h0[1].wait()
                _chunk(k0, tsb0, tnb0, base)
                h1[0].wait()
                h1[1].wait()
                _chunk(k1, tsb1, tnb1, base)
                return 0
            lax.fori_loop(0, _GRP // (2 * _CH), _pair, 0)
            return 0
        lax.fori_loop(0, ng, _group, 0)

        pltpu.sync_copy(tile, acc_hbm.at[wid])
        pltpu.sync_copy(denb, den_hbm.at[wid])
        return 0
    lax.fori_loop(0, _WPT, _pass, 0)


@jax.jit
def _sc_edge(ts, tn, s2, n2, w2g, est):
    f32 = jnp.float32
    return pl.kernel(
        _sc_edge_body,
        out_type=[jax.ShapeDtypeStruct((_NV, _NPW, _HID), f32),
                  jax.ShapeDtypeStruct((_NV, _DR, 16), f32)],
        mesh=plsc.VectorSubcoreMesh(core_axis_name="c", subcore_axis_name="s",
                                    num_cores=_NC, num_subcores=_NS),
        compiler_params=pltpu.CompilerParams(needs_layout_passes=False),
        scratch_types=[
            pltpu.VMEM((_GRP // _CH, _CH), jnp.int32),
            pltpu.VMEM((_GRP // _CH, _CH), jnp.int32),
            pltpu.VMEM((_CH, 2 * _HID), f32),
            pltpu.VMEM((_CH, 2 * _HID), f32),
            pltpu.VMEM((_CH, _TNW), f32),
            pltpu.VMEM((_CH, _TNW), f32),
            pltpu.VMEM((_NPW, _HID), f32),
            pltpu.VMEM((_DR, 16), f32),
            pltpu.VMEM((_HID + 16,), f32),
            pltpu.VMEM((1, 16), jnp.int32),
            pltpu.VMEM((_CH * _CH,), f32),
            pltpu.SemaphoreType.DMA((4,)),
        ],
    )(ts, tn, s2, n2, w2g, est)


def _mm_body(x_ref, w_ref, b_ref, o_ref):
    o_ref[...] = (
        jnp.dot(x_ref[...], w_ref[...], preferred_element_type=jnp.float32)
        + b_ref[...]
    )


@functools.partial(jax.jit, static_argnames=("bn",))
def _mm(x, w, b, bn=400):
    r, k = x.shape
    l = w.shape[1]
    assert r % bn == 0, (r, bn)
    return pl.pallas_call(
        _mm_body,
        grid=(r // bn,),
        in_specs=[
            pl.BlockSpec((bn, k), lambda i: (i, 0)),
            pl.BlockSpec((k, l), lambda i: (0, 0)),
            pl.BlockSpec((1, l), lambda i: (0, 0)),
        ],
        out_specs=pl.BlockSpec((bn, l), lambda i: (i, 0)),
        out_shape=jax.ShapeDtypeStruct((r, l), jnp.float32),
    )(x, w, b.reshape(1, l))


def _leaky(x):
    return jnp.where(x > 0, x, 0.01 * x)


def kernel(elem_weights, elem_fea, emb_W, emb_b, g_gate_W1, g_gate_b1, g_gate_W2, g_gate_b2, g_msg_W1, g_msg_b1, g_msg_W2, g_msg_b2, g_pow, c_gate_W1, c_gate_b1, c_gate_W2, c_gate_b2, c_msg_W1, c_msg_b1, c_msg_W2, c_msg_b2, c_pow, self_fea_idx, nbr_fea_idx, cry_elem_idx):
    w1 = elem_weights[:, 0]
    logw = jnp.log(w1)

    s_i = self_fea_idx.astype(jnp.int32)
    n_i = nbr_fea_idx.astype(jnp.int32)
    s2 = jnp.concatenate([s_i, jnp.zeros((_MP - _M,), jnp.int32)]
                         ).reshape(_MP // _GRP, _GRP // _CH, _CH)
    n2 = jnp.concatenate([n_i, jnp.zeros((_MP - _M,), jnp.int32)]
                         ).reshape(_MP // _GRP, _GRP // _CH, _CH)
    esr = jnp.searchsorted(s_i, jnp.arange(_NV + 1) * _NPW).astype(jnp.int32)
    est = (jnp.zeros((_NV, 16), jnp.int32)
           .at[:, 0].set(esr[:-1]).at[:, 1].set(esr[1:])
           ).reshape(_NV, 1, 16)

    emb = _mm(elem_fea, emb_W, emb_b)
    fea = jnp.concatenate([emb, elem_weights], axis=1)

    for i in range(_NG):
        wcat, bcat = [], []
        for h in range(_HEADS):
            j = i * _HEADS + h
            wcat += [g_gate_W1[j][:_FEA], g_msg_W1[j][:_FEA],
                     g_gate_W1[j][_FEA:], g_msg_W1[j][_FEA:]]
            bcat += [g_gate_b1[j], g_msg_b1[j],
                     jnp.zeros_like(g_gate_b1[j]), jnp.zeros_like(g_msg_b1[j])]
        tables = _mm(fea, jnp.concatenate(wcat, axis=1), jnp.concatenate(bcat))

        parts = []
        for h in range(_HEADS):
            j = i * _HEADS + h
            ts = tables[:, h * 4 * _HID:(h * 4 + 2) * _HID]
            lwj = (g_pow[j] * logw)[:, None]
            tn = jnp.concatenate(
                [tables[:, (h * 4 + 2) * _HID:(h * 4 + 4) * _HID],
                 jnp.minimum(lwj, 0.0), jnp.maximum(lwj, 0.0),
                 jnp.zeros((_N, 126))], axis=1)
            w2g = jnp.concatenate(
                [g_gate_W2[j][:, 0], jnp.array([100.0, 1.0]),
                 jnp.zeros((14,))])
            acc_p, den_p = _sc_edge(ts, tn, s2, n2, w2g, est)
            acc = acc_p.reshape(_NOUT, _HID)[:_N]
            den = den_p.reshape(_NOUT)[:_N]
            dsafe = jnp.where(den > 0, den, 1.0)
            node = acc / dsafe[:, None]
            gsum = jnp.where(den > 0, 1.0, 0.0)
            parts += [node, gsum[:, None]]

        wrows, x_parts = [], []
        for h in range(_HEADS):
            j = i * _HEADS + h
            wrows += [g_msg_W2[j] / _HEADS, g_msg_b2[j][None, :] / _HEADS]
        wrows.append(jnp.eye(_FEA))
        x = jnp.concatenate(parts + [fea], axis=1)
        wbig = jnp.concatenate(wrows, axis=0)
        fea = _mm(x, wbig, jnp.zeros((_FEA,)))

    wcat, bcat = [], []
    for h in range(_HEADS):
        wcat += [c_gate_W1[h], c_msg_W1[h]]
        bcat += [c_gate_b1[h], c_msg_b1[h]]
    hid = _leaky(_mm(fea, jnp.concatenate(wcat, axis=1), jnp.concatenate(bcat)))
    wbd = jnp.zeros((2 * _HEADS * _HID, _HEADS))
    for h in range(_HEADS):
        wbd = wbd.at[2 * h * _HID:(2 * h + 1) * _HID, h].set(c_gate_W2[h][:, 0])
    g3 = _mm(hid, wbd, jnp.zeros((_HEADS,)))
    u3 = jnp.exp(g3 + logw[:, None] * c_pow[None, :])

    parts = []
    for h in range(_HEADS):
        hm = hid[:, (2 * h + 1) * _HID:(2 * h + 2) * _HID]
        acc = jax.ops.segment_sum(u3[:, h:h + 1] * hm, cry_elem_idx,
                                  num_segments=_C)
        den = jax.ops.segment_sum(u3[:, h], cry_elem_idx, num_segments=_C)
        dsafe = jnp.where(den > 0, den, 1.0)
        parts += [acc / dsafe[:, None],
                  jnp.where(den > 0, 1.0, 0.0)[:, None]]
    wrows = []
    for h in range(_HEADS):
        wrows += [c_msg_W2[h] / _HEADS, c_msg_b2[h][None, :] / _HEADS]
    xc = jnp.concatenate(parts, axis=1)
    wc = jnp.concatenate(wrows, axis=0)
    return _mm(xc, wc, jnp.zeros((_FEA,)))


# speedup vs baseline: 1.9872x; 1.9872x over previous
import functools

import jax
import jax.numpy as jnp
from jax import lax
from jax.experimental import pallas as pl
from jax.experimental.pallas import tpu as pltpu
from jax.experimental.pallas import tpu_sc as plsc

_N = 10000
_M = 320000
_C = 2000
_FEA = 128
_NG = 3
_HEADS = 3
_HID = 256

_NC = 2
_NS = 16
_NW = _NC * _NS
_WPT = 1
_NV = _NW * _WPT
_NPW = 320
_NOUT = _NV * _NPW
_CH = 16
_GRP = 256
_MP = _M + 2 * _GRP
_TNW = 2 * _HID + 128
_DR = _NPW // 16


def _sc_edge_body(ts_hbm, tn_hbm, s2_hbm, n2_hbm, w2g_hbm, est_hbm,
                  acc_hbm, den_hbm,
                  sblk, nblk, tsb0, tsb1, tnb0, tnb1, tile, denb, w2gb,
                  estb, gbuf, sems):
    cid = lax.axis_index("c")
    sid = lax.axis_index("s")
    w = sid * _NC + cid

    pltpu.sync_copy(w2g_hbm, w2gb)
    zv = jnp.zeros((16,), jnp.float32)
    lanes = lax.iota(jnp.int32, 16)

    def _pass(sp, _0):
        wid = w * _WPT + sp
        nbase = wid * _NPW
        pltpu.sync_copy(est_hbm.at[wid], estb)
        ev = estb[0, pl.ds(0, 16)]
        e0 = ev[0]
        e1 = ev[1]

        def _zrow(r, _):
            for k2 in range(_HID // 16):
                tile[r, pl.ds(k2 * 16, 16)] = zv
            return 0
        lax.fori_loop(0, _NPW, _zrow, 0)

        def _zden(r, _):
            denb[r, pl.ds(0, 16)] = zv
            return 0
        lax.fori_loop(0, _DR, _zden, 0)

        e0a = (e0 // _GRP) * _GRP
        ng = (e1 - e0a + _GRP - 1) // _GRP

        def _chunk(k, tsr, tnr, base):
            cbase = base + k * _CH
            sv = sblk[k, pl.ds(0, 16)]
            for e in range(_CH):
                lv = tnr[e, pl.ds(2 * _HID, 16)]
                lv = jnp.where(lv > 0, lv, 0.01 * lv)

                def _dot(k2, vp):
                    a = (tsr[e, pl.ds(k2 * 16, 16)]
                         + tnr[e, pl.ds(k2 * 16, 16)])
                    a = jnp.where(a > 0, a, 0.01 * a)
                    return vp + a * w2gb[pl.ds(k2 * 16, 16)]
                vp = lax.fori_loop(0, _HID // 16, _dot,
                                   lv * w2gb[pl.ds(_HID, 16)])
                gbuf[pl.ds(e * _CH, _CH)] = vp

            gvec = zv
            for j in range(_CH):
                gvec = gvec + plsc.load_gather(gbuf, [lanes * _CH + j])

            mask = (lanes + cbase >= e0) & (lanes + cbase < e1)
            uv = jnp.where(mask, jnp.exp(gvec), 0.0)

            for e in range(_CH):
                u = uv[e]
                sl = jnp.clip(sv[e] - nbase, 0, _NPW - 1)

                def _acc(k2, _2):
                    bvec = (tsr[e, pl.ds(_HID + k2 * 16, 16)]
                            + tnr[e, pl.ds(_HID + k2 * 16, 16)])
                    bvec = jnp.where(bvec > 0, bvec, 0.01 * bvec)
                    tile[sl, pl.ds(k2 * 16, 16)] = (
                        tile[sl, pl.ds(k2 * 16, 16)] + u * bvec)
                    return 0
                lax.fori_loop(0, _HID // 16, _acc, 0)
                dr = sl // 16
                dhot = jnp.where(lanes == sl % 16, u, 0.0)
                denb[dr, pl.ds(0, 16)] = denb[dr, pl.ds(0, 16)] + dhot

        def _group(g, _):
            base = e0a + g * _GRP
            gi = base // _GRP
            pltpu.sync_copy(s2_hbm.at[gi], sblk)
            pltpu.sync_copy(n2_hbm.at[gi], nblk)

            def _pair(cp, _2):
                k0 = 2 * cp
                k1 = k0 + 1
                h0 = (pltpu.async_copy(ts_hbm.at[sblk.at[k0]], tsb0,
                                       sems.at[0]),
                      pltpu.async_copy(tn_hbm.at[nblk.at[k0]], tnb0,
                                       sems.at[1]))
                h1 = (pltpu.async_copy(ts_hbm.at[sblk.at[k1]], tsb1,
                                       sems.at[2]),
                      pltpu.async_copy(tn_hbm.at[nblk.at[k1]], tnb1,
                                       sems.at[3]))
                h0[0].wait()
                h0[1].wait()
                _chunk(k0, tsb0, tnb0, base)
                h1[0].wait()
                h1[1].wait()
                _chunk(k1, tsb1, tnb1, base)
                return 0
            lax.fori_loop(0, _GRP // (2 * _CH), _pair, 0)
            return 0
        lax.fori_loop(0, ng, _group, 0)

        pltpu.sync_copy(tile, acc_hbm.at[wid])
        pltpu.sync_copy(denb, den_hbm.at[wid])
        return 0
    lax.fori_loop(0, _WPT, _pass, 0)


@jax.jit
def _sc_edge(ts, tn, s2, n2, w2g, est):
    f32 = jnp.float32
    return pl.kernel(
        _sc_edge_body,
        out_type=[jax.ShapeDtypeStruct((_NV, _NPW, _HID), f32),
                  jax.ShapeDtypeStruct((_NV, _DR, 16), f32)],
        mesh=plsc.VectorSubcoreMesh(core_axis_name="c", subcore_axis_name="s",
                                    num_cores=_NC, num_subcores=_NS),
        compiler_params=pltpu.CompilerParams(needs_layout_passes=False),
        scratch_types=[
            pltpu.VMEM((_GRP // _CH, _CH), jnp.int32),
            pltpu.VMEM((_GRP // _CH, _CH), jnp.int32),
            pltpu.VMEM((_CH, 2 * _HID), f32),
            pltpu.VMEM((_CH, 2 * _HID), f32),
            pltpu.VMEM((_CH, _TNW), f32),
            pltpu.VMEM((_CH, _TNW), f32),
            pltpu.VMEM((_NPW, _HID), f32),
            pltpu.VMEM((_DR, 16), f32),
            pltpu.VMEM((_HID + 16,), f32),
            pltpu.VMEM((1, 16), jnp.int32),
            pltpu.VMEM((_CH * _CH,), f32),
            pltpu.SemaphoreType.DMA((4,)),
        ],
    )(ts, tn, s2, n2, w2g, est)


def _mm_body(x_ref, w_ref, b_ref, o_ref):
    o_ref[...] = (
        jnp.dot(x_ref[...], w_ref[...], preferred_element_type=jnp.float32)
        + b_ref[...]
    )


@functools.partial(jax.jit, static_argnames=("bn",))
def _mm(x, w, b, bn=400):
    r, k = x.shape
    l = w.shape[1]
    assert r % bn == 0, (r, bn)
    return pl.pallas_call(
        _mm_body,
        grid=(r // bn,),
        in_specs=[
            pl.BlockSpec((bn, k), lambda i: (i, 0)),
            pl.BlockSpec((k, l), lambda i: (0, 0)),
            pl.BlockSpec((1, l), lambda i: (0, 0)),
        ],
        out_specs=pl.BlockSpec((bn, l), lambda i: (i, 0)),
        out_shape=jax.ShapeDtypeStruct((r, l), jnp.float32),
    )(x, w, b.reshape(1, l))


def _leaky(x):
    return jnp.where(x > 0, x, 0.01 * x)


def kernel(elem_weights, elem_fea, emb_W, emb_b, g_gate_W1, g_gate_b1, g_gate_W2, g_gate_b2, g_msg_W1, g_msg_b1, g_msg_W2, g_msg_b2, g_pow, c_gate_W1, c_gate_b1, c_gate_W2, c_gate_b2, c_msg_W1, c_msg_b1, c_msg_W2, c_msg_b2, c_pow, self_fea_idx, nbr_fea_idx, cry_elem_idx):
    w1 = elem_weights[:, 0]
    logw = jnp.log(w1)

    s_i = self_fea_idx.astype(jnp.int32)
    n_i = nbr_fea_idx.astype(jnp.int32)
    s2 = jnp.concatenate([s_i, jnp.zeros((_MP - _M,), jnp.int32)]
                         ).reshape(_MP // _GRP, _GRP // _CH, _CH)
    n2 = jnp.concatenate([n_i, jnp.zeros((_MP - _M,), jnp.int32)]
                         ).reshape(_MP // _GRP, _GRP // _CH, _CH)
    esr = jnp.searchsorted(s_i, jnp.arange(_NV + 1) * _NPW).astype(jnp.int32)
    est = (jnp.zeros((_NV, 16), jnp.int32)
           .at[:, 0].set(esr[:-1]).at[:, 1].set(esr[1:])
           ).reshape(_NV, 1, 16)

    emb = _mm(elem_fea, emb_W, emb_b)
    fea = jnp.concatenate([emb, elem_weights], axis=1)

    for i in range(_NG):
        wcat, bcat = [], []
        for h in range(_HEADS):
            j = i * _HEADS + h
            wcat += [g_gate_W1[j][:_FEA], g_msg_W1[j][:_FEA],
                     g_gate_W1[j][_FEA:], g_msg_W1[j][_FEA:]]
            bcat += [g_gate_b1[j], g_msg_b1[j],
                     jnp.zeros_like(g_gate_b1[j]), jnp.zeros_like(g_msg_b1[j])]
        tables = _mm(fea, jnp.concatenate(wcat, axis=1), jnp.concatenate(bcat))

        parts = []
        for h in range(_HEADS):
            j = i * _HEADS + h
            ts = tables[:, h * 4 * _HID:(h * 4 + 2) * _HID]
            lwj = (g_pow[j] * logw)[:, None]
            tn = jnp.concatenate(
                [tables[:, (h * 4 + 2) * _HID:(h * 4 + 4) * _HID],
                 jnp.minimum(lwj, 0.0), jnp.maximum(lwj, 0.0),
                 jnp.zeros((_N, 126))], axis=1)
            w2g = jnp.concatenate(
                [g_gate_W2[j][:, 0], jnp.array([100.0, 1.0]),
                 jnp.zeros((14,))])
            acc_p, den_p = _sc_edge(ts, tn, s2, n2, w2g, est)
            acc = acc_p.reshape(_NOUT, _HID)[:_N]
            den = den_p.reshape(_NOUT)[:_N]
            dsafe = jnp.where(den > 0, den, 1.0)
            node = acc / dsafe[:, None]
            gsum = jnp.where(den > 0, 1.0, 0.0)
            parts += [node, gsum[:, None]]

        wrows, x_parts = [], []
        for h in range(_HEADS):
            j = i * _HEADS + h
            wrows += [g_msg_W2[j] / _HEADS, g_msg_b2[j][None, :] / _HEADS]
        wrows.append(jnp.eye(_FEA))
        x = jnp.concatenate(parts + [fea], axis=1)
        wbig = jnp.concatenate(wrows, axis=0)
        fea = _mm(x, wbig, jnp.zeros((_FEA,)))

    wcat, bcat = [], []
    for h in range(_HEADS):
        wcat += [c_gate_W1[h], c_msg_W1[h]]
        bcat += [c_gate_b1[h], c_msg_b1[h]]
    hid = _leaky(_mm(fea, jnp.concatenate(wcat, axis=1), jnp.concatenate(bcat)))
    wbd = jnp.zeros((2 * _HEADS * _HID, _HEADS))
    for h in range(_HEADS):
        wbd = wbd.at[2 * h * _HID:(2 * h + 1) * _HID, h].set(c_gate_W2[h][:, 0])
    g3 = _mm(hid, wbd, jnp.zeros((_HEADS,)))
    u3 = jnp.exp(g3 + logw[:, None] * c_pow[None, :])

    parts = []
    for h in range(_HEADS):
        hm = hid[:, (2 * h + 1) * _HID:(2 * h + 2) * _HID]
        acc = jax.ops.segment_sum(u3[:, h:h + 1] * hm, cry_elem_idx,
                                  num_segments=_C)
        den = jax.ops.segment_sum(u3[:, h], cry_elem_idx, num_segments=_C)
        dsafe = jnp.where(den > 0, den, 1.0)
        parts += [acc / dsafe[:, None],
                  jnp.where(den > 0, 1.0, 0.0)[:, None]]
    wrows = []
    for h in range(_HEADS):
        wrows += [c_msg_W2[h] / _HEADS, c_msg_b2[h][None, :] / _HEADS]
    xc = jnp.concatenate(parts, axis=1)
    wc = jnp.concatenate(wrows, axis=0)
    return _mm(xc, wc, jnp.zeros((_FEA,)))


# speedup vs baseline: 2.4576x; 1.2367x over previous
import functools

import jax
import jax.numpy as jnp
from jax import lax
from jax.experimental import pallas as pl
from jax.experimental.pallas import tpu as pltpu
from jax.experimental.pallas import tpu_sc as plsc

_N = 10000
_M = 320000
_C = 2000
_FEA = 128
_NG = 3
_HEADS = 3
_HID = 256

_NC = 2
_NS = 16
_NW = _NC * _NS
_WPT = 1
_NV = _NW * _WPT
_NPW = 320
_NOUT = _NV * _NPW
_CH = 16
_GRP = 256
_MP = _M + 2 * _GRP
_TNW = 2 * _HID + 128
_DR = _NPW // 16


def _sc_edge_body(ts_hbm, tn_hbm, s2_hbm, n2_hbm, w2g_hbm, est_hbm,
                  acc_hbm, den_hbm,
                  sblk, nblk, tsb0, tsb1, tnb0, tnb1, tile, denb, w2gb,
                  estb, gbuf, sems):
    cid = lax.axis_index("c")
    sid = lax.axis_index("s")
    w = sid * _NC + cid

    pltpu.sync_copy(w2g_hbm, w2gb)
    zv = jnp.zeros((16,), jnp.float32)
    lanes = lax.iota(jnp.int32, 16)

    def _pass(sp, _0):
        wid = w * _WPT + sp
        nbase = wid * _NPW
        pltpu.sync_copy(est_hbm.at[wid], estb)
        ev = estb[0, pl.ds(0, 16)]
        e0 = ev[0]
        e1 = ev[1]

        def _zrow(r, _):
            for k2 in range(_HID // 16):
                tile[r, pl.ds(k2 * 16, 16)] = zv
            return 0
        lax.fori_loop(0, _NPW, _zrow, 0)

        def _zden(r, _):
            denb[r, pl.ds(0, 16)] = zv
            return 0
        lax.fori_loop(0, _DR, _zden, 0)

        e0a = (e0 // _GRP) * _GRP
        ng = (e1 - e0a + _GRP - 1) // _GRP

        def _chunk(k, tsr, tnr, base):
            cbase = base + k * _CH
            sv = sblk[k, pl.ds(0, 16)]
            for e in range(_CH):
                lv = tnr[e, pl.ds(2 * _HID, 16)]
                lv = jnp.where(lv > 0, lv, 0.01 * lv)

                @plsc.parallel_loop(0, _HID // 16, unroll=4,
                                    carry=lv * w2gb[pl.ds(_HID, 16)])
                def vp(k2, vp):
                    a = (tsr[e, pl.ds(k2 * 16, 16)]
                         + tnr[e, pl.ds(k2 * 16, 16)])
                    a = jnp.where(a > 0, a, 0.01 * a)
                    return vp + a * w2gb[pl.ds(k2 * 16, 16)]
                gbuf[pl.ds(e * _CH, _CH)] = vp

            gvec = zv
            for j in range(_CH):
                gvec = gvec + plsc.load_gather(gbuf, [lanes * _CH + j])

            mask = (lanes + cbase >= e0) & (lanes + cbase < e1)
            uv = jnp.where(mask, jnp.exp(gvec), 0.0)

            for e in range(_CH):
                u = uv[e]
                sl = jnp.clip(sv[e] - nbase, 0, _NPW - 1)

                @plsc.parallel_loop(0, _HID // 16, unroll=4)
                def _acc(k2):
                    bvec = (tsr[e, pl.ds(_HID + k2 * 16, 16)]
                            + tnr[e, pl.ds(_HID + k2 * 16, 16)])
                    bvec = jnp.where(bvec > 0, bvec, 0.01 * bvec)
                    tile[sl, pl.ds(k2 * 16, 16)] = (
                        tile[sl, pl.ds(k2 * 16, 16)] + u * bvec)
                dr = sl // 16
                dhot = jnp.where(lanes == sl % 16, u, 0.0)
                denb[dr, pl.ds(0, 16)] = denb[dr, pl.ds(0, 16)] + dhot

        def _group(g, _):
            base = e0a + g * _GRP
            gi = base // _GRP
            pltpu.sync_copy(s2_hbm.at[gi], sblk)
            pltpu.sync_copy(n2_hbm.at[gi], nblk)

            def _pair(cp, _2):
                k0 = 2 * cp
                k1 = k0 + 1
                h0 = (pltpu.async_copy(ts_hbm.at[sblk.at[k0]], tsb0,
                                       sems.at[0]),
                      pltpu.async_copy(tn_hbm.at[nblk.at[k0]], tnb0,
                                       sems.at[1]))
                h1 = (pltpu.async_copy(ts_hbm.at[sblk.at[k1]], tsb1,
                                       sems.at[2]),
                      pltpu.async_copy(tn_hbm.at[nblk.at[k1]], tnb1,
                                       sems.at[3]))
                h0[0].wait()
                h0[1].wait()
                _chunk(k0, tsb0, tnb0, base)
                h1[0].wait()
                h1[1].wait()
                _chunk(k1, tsb1, tnb1, base)
                return 0
            lax.fori_loop(0, _GRP // (2 * _CH), _pair, 0)
            return 0
        lax.fori_loop(0, ng, _group, 0)

        pltpu.sync_copy(tile, acc_hbm.at[wid])
        pltpu.sync_copy(denb, den_hbm.at[wid])
        return 0
    lax.fori_loop(0, _WPT, _pass, 0)


@jax.jit
def _sc_edge(ts, tn, s2, n2, w2g, est):
    f32 = jnp.float32
    return pl.kernel(
        _sc_edge_body,
        out_type=[jax.ShapeDtypeStruct((_NV, _NPW, _HID), f32),
                  jax.ShapeDtypeStruct((_NV, _DR, 16), f32)],
        mesh=plsc.VectorSubcoreMesh(core_axis_name="c", subcore_axis_name="s",
                                    num_cores=_NC, num_subcores=_NS),
        compiler_params=pltpu.CompilerParams(needs_layout_passes=False),
        scratch_types=[
            pltpu.VMEM((_GRP // _CH, _CH), jnp.int32),
            pltpu.VMEM((_GRP // _CH, _CH), jnp.int32),
            pltpu.VMEM((_CH, 2 * _HID), f32),
            pltpu.VMEM((_CH, 2 * _HID), f32),
            pltpu.VMEM((_CH, _TNW), f32),
            pltpu.VMEM((_CH, _TNW), f32),
            pltpu.VMEM((_NPW, _HID), f32),
            pltpu.VMEM((_DR, 16), f32),
            pltpu.VMEM((_HID + 16,), f32),
            pltpu.VMEM((1, 16), jnp.int32),
            pltpu.VMEM((_CH * _CH,), f32),
            pltpu.SemaphoreType.DMA((4,)),
        ],
    )(ts, tn, s2, n2, w2g, est)


def _mm_body(x_ref, w_ref, b_ref, o_ref):
    o_ref[...] = (
        jnp.dot(x_ref[...], w_ref[...], preferred_element_type=jnp.float32)
        + b_ref[...]
    )


@functools.partial(jax.jit, static_argnames=("bn",))
def _mm(x, w, b, bn=400):
    r, k = x.shape
    l = w.shape[1]
    assert r % bn == 0, (r, bn)
    return pl.pallas_call(
        _mm_body,
        grid=(r // bn,),
        in_specs=[
            pl.BlockSpec((bn, k), lambda i: (i, 0)),
            pl.BlockSpec((k, l), lambda i: (0, 0)),
            pl.BlockSpec((1, l), lambda i: (0, 0)),
        ],
        out_specs=pl.BlockSpec((bn, l), lambda i: (i, 0)),
        out_shape=jax.ShapeDtypeStruct((r, l), jnp.float32),
    )(x, w, b.reshape(1, l))


def _leaky(x):
    return jnp.where(x > 0, x, 0.01 * x)


def kernel(elem_weights, elem_fea, emb_W, emb_b, g_gate_W1, g_gate_b1, g_gate_W2, g_gate_b2, g_msg_W1, g_msg_b1, g_msg_W2, g_msg_b2, g_pow, c_gate_W1, c_gate_b1, c_gate_W2, c_gate_b2, c_msg_W1, c_msg_b1, c_msg_W2, c_msg_b2, c_pow, self_fea_idx, nbr_fea_idx, cry_elem_idx):
    w1 = elem_weights[:, 0]
    logw = jnp.log(w1)

    s_i = self_fea_idx.astype(jnp.int32)
    n_i = nbr_fea_idx.astype(jnp.int32)
    s2 = jnp.concatenate([s_i, jnp.zeros((_MP - _M,), jnp.int32)]
                         ).reshape(_MP // _GRP, _GRP // _CH, _CH)
    n2 = jnp.concatenate([n_i, jnp.zeros((_MP - _M,), jnp.int32)]
                         ).reshape(_MP // _GRP, _GRP // _CH, _CH)
    esr = jnp.searchsorted(s_i, jnp.arange(_NV + 1) * _NPW).astype(jnp.int32)
    est = (jnp.zeros((_NV, 16), jnp.int32)
           .at[:, 0].set(esr[:-1]).at[:, 1].set(esr[1:])
           ).reshape(_NV, 1, 16)

    emb = _mm(elem_fea, emb_W, emb_b)
    fea = jnp.concatenate([emb, elem_weights], axis=1)

    for i in range(_NG):
        wcat, bcat = [], []
        for h in range(_HEADS):
            j = i * _HEADS + h
            wcat += [g_gate_W1[j][:_FEA], g_msg_W1[j][:_FEA],
                     g_gate_W1[j][_FEA:], g_msg_W1[j][_FEA:]]
            bcat += [g_gate_b1[j], g_msg_b1[j],
                     jnp.zeros_like(g_gate_b1[j]), jnp.zeros_like(g_msg_b1[j])]
        tables = _mm(fea, jnp.concatenate(wcat, axis=1), jnp.concatenate(bcat))

        parts = []
        for h in range(_HEADS):
            j = i * _HEADS + h
            ts = tables[:, h * 4 * _HID:(h * 4 + 2) * _HID]
            lwj = (g_pow[j] * logw)[:, None]
            tn = jnp.concatenate(
                [tables[:, (h * 4 + 2) * _HID:(h * 4 + 4) * _HID],
                 jnp.minimum(lwj, 0.0), jnp.maximum(lwj, 0.0),
                 jnp.zeros((_N, 126))], axis=1)
            w2g = jnp.concatenate(
                [g_gate_W2[j][:, 0], jnp.array([100.0, 1.0]),
                 jnp.zeros((14,))])
            acc_p, den_p = _sc_edge(ts, tn, s2, n2, w2g, est)
            acc = acc_p.reshape(_NOUT, _HID)[:_N]
            den = den_p.reshape(_NOUT)[:_N]
            dsafe = jnp.where(den > 0, den, 1.0)
            node = acc / dsafe[:, None]
            gsum = jnp.where(den > 0, 1.0, 0.0)
            parts += [node, gsum[:, None]]

        wrows, x_parts = [], []
        for h in range(_HEADS):
            j = i * _HEADS + h
            wrows += [g_msg_W2[j] / _HEADS, g_msg_b2[j][None, :] / _HEADS]
        wrows.append(jnp.eye(_FEA))
        x = jnp.concatenate(parts + [fea], axis=1)
        wbig = jnp.concatenate(wrows, axis=0)
        fea = _mm(x, wbig, jnp.zeros((_FEA,)))

    wcat, bcat = [], []
    for h in range(_HEADS):
        wcat += [c_gate_W1[h], c_msg_W1[h]]
        bcat += [c_gate_b1[h], c_msg_b1[h]]
    hid = _leaky(_mm(fea, jnp.concatenate(wcat, axis=1), jnp.concatenate(bcat)))
    wbd = jnp.zeros((2 * _HEADS * _HID, _HEADS))
    for h in range(_HEADS):
        wbd = wbd.at[2 * h * _HID:(2 * h + 1) * _HID, h].set(c_gate_W2[h][:, 0])
    g3 = _mm(hid, wbd, jnp.zeros((_HEADS,)))
    u3 = jnp.exp(g3 + logw[:, None] * c_pow[None, :])

    parts = []
    for h in range(_HEADS):
        hm = hid[:, (2 * h + 1) * _HID:(2 * h + 2) * _HID]
        acc = jax.ops.segment_sum(u3[:, h:h + 1] * hm, cry_elem_idx,
                                  num_segments=_C)
        den = jax.ops.segment_sum(u3[:, h], cry_elem_idx, num_segments=_C)
        dsafe = jnp.where(den > 0, den, 1.0)
        parts += [acc / dsafe[:, None],
                  jnp.where(den > 0, 1.0, 0.0)[:, None]]
    wrows = []
    for h in range(_HEADS):
        wrows += [c_msg_W2[h] / _HEADS, c_msg_b2[h][None, :] / _HEADS]
    xc = jnp.concatenate(parts, axis=1)
    wc = jnp.concatenate(wrows, axis=0)
    return _mm(xc, wc, jnp.zeros((_FEA,)))
